# baseline (speedup 1.0000x reference)
import functools

import jax
import jax.numpy as jnp
from jax.experimental import pallas as pl
from jax.experimental.pallas import tpu as pltpu

N = 10000
E = 160000
NUM_GRAPHS = 64
NUM_CLASSES = 40
EMBED = 32
L0 = 128
L1 = 512
L2 = 2048
L3 = 4096


def _leaky(v, s):
    return jnp.where(v >= 0, v, s * v)


def _ln(v, g, b):
    m = jnp.mean(v, axis=-1, keepdims=True)
    var = jnp.mean((v - m) ** 2, axis=-1, keepdims=True)
    return (v - m) / jnp.sqrt(var + 1e-5) * g + b


def _dot(a, b):
    return jnp.dot(a, b, preferred_element_type=jnp.float32)


def _node_enc_body(x_ref, emby_ref, embx_ref, embp_ref, smw_ref, smb_ref,
                   how_ref, hob_ref, orw_ref, orb_ref, fmw_ref, fmb_ref,
                   g0_ref, b0_ref, feats_ref, h_ref):
    x = x_ref[...]

    def onehot(col, k):
        i = jax.lax.broadcasted_iota(jnp.int32, (1, k), 1)
        return (col.astype(jnp.int32) == i).astype(jnp.float32)

    cy = _dot(onehot(x[:, 0:1], 18), emby_ref[...])
    cx = _dot(onehot(x[:, 1:2], 11), embx_ref[...])
    coords = _leaky(_dot(jnp.concatenate([cy, cx], axis=1), smw_ref[...])
                    + smb_ref[...], 0.01)
    positions = _leaky(_dot(onehot(x[:, 2:3], 140), embp_ref[...]), 0.01)
    ht = x[:, 3:8]
    ht_e = (_dot(ht, how_ref[...]) + hob_ref[...]) / (
        jnp.sum(ht, axis=1, keepdims=True) + 1e-8)
    ori = x[:, 8:16]
    ori_e = (_dot(ori, orw_ref[...]) + orb_ref[...]) / (
        jnp.sum(ori, axis=1, keepdims=True) + 1e-8)
    feats = jnp.concatenate([coords, ori_e, ht_e, positions], axis=1)
    feats_ref[...] = feats
    h_ref[...] = jax.nn.relu(_ln(_dot(feats, fmw_ref[...]) + fmb_ref[...],
                                 g0_ref[...], b0_ref[...]))


def _node_encoder(x, p):
    outs = pl.pallas_call(
        _node_enc_body,
        out_shape=[jax.ShapeDtypeStruct((N, L0), jnp.float32),
                   jax.ShapeDtypeStruct((N, L0), jnp.float32)],
    )(x, p["emb_y"], p["emb_x"], p["emb_pos"],
      p["smoosh"]["W"], p["smoosh"]["b"][None, :],
      p["hold"]["W"], p["hold"]["b"][None, :],
      p["orient"]["W"], p["orient"]["b"][None, :],
      p["feat_mix"]["W"], p["feat_mix"]["b"][None, :],
      p["ln0"]["g"][None, :], p["ln0"]["b"][None, :])
    return outs[0], outs[1]


def _edge_mlp_body(fa_ref, fb_ref, w1a_ref, w1b_ref, b1_ref, g1_ref, b1n_ref,
                   w2_ref, b2_ref, g0_ref, b0n_ref, w_ref):
    u = (_dot(fa_ref[...][:, :96], w1a_ref[...])
         + _dot(fb_ref[...][:, :96], w1b_ref[...]) + b1_ref[...])
    u = jax.nn.relu(_ln(u, g1_ref[...], b1n_ref[...]))
    v = _dot(u, w2_ref[...]) + b2_ref[...]
    w_ref[...] = _leaky(_ln(v, g0_ref[...], b0n_ref[...]), 0.01)


def _edge_mlp(fa, fb, p, be=2000):
    grid = (E // be,)
    espec = pl.BlockSpec((be, L0), lambda i: (i, 0))
    wspec = lambda shape: pl.BlockSpec(shape, lambda i: (0, 0))
    w1 = p["ew1"]["W"]
    return pl.pallas_call(
        _edge_mlp_body,
        grid=grid,
        in_specs=[espec, espec,
                  wspec((96, L1)), wspec((96, L1)), wspec((1, L1)),
                  wspec((1, L1)), wspec((1, L1)),
                  wspec((L1, L0)), wspec((1, L0)),
                  wspec((1, L0)), wspec((1, L0))],
        out_specs=espec,
        out_shape=jax.ShapeDtypeStruct((E, L0), jnp.float32),
    )(fa, fb, w1[:96], w1[96:], p["ew1"]["b"][None, :],
      p["ln1"]["g"][None, :], p["ln1"]["b"][None, :],
      p["ew2"]["W"], p["ew2"]["b"][None, :],
      p["ln0"]["g"][None, :], p["ln0"]["b"][None, :])


def _dense2_body(h_ref, wl_ref, bl_ref, wr_ref, br_ref, xl_ref, xr_ref):
    h = h_ref[...]
    xl_ref[...] = _dot(h, wl_ref[...]) + bl_ref[...]
    xr_ref[...] = _dot(h, wr_ref[...]) + br_ref[...]


def _dense2(h, c, i_dim, o_dim, bn, bo):
    grid = (N // bn, o_dim // bo)
    ospec = pl.BlockSpec((bn, bo), lambda i, j: (i, j))
    return pl.pallas_call(
        _dense2_body,
        grid=grid,
        in_specs=[pl.BlockSpec((bn, i_dim), lambda i, j: (i, 0)),
                  pl.BlockSpec((i_dim, bo), lambda i, j: (0, j)),
                  pl.BlockSpec((1, bo), lambda i, j: (0, j)),
                  pl.BlockSpec((i_dim, bo), lambda i, j: (0, j)),
                  pl.BlockSpec((1, bo), lambda i, j: (0, j))],
        out_specs=[ospec, ospec],
        out_shape=[jax.ShapeDtypeStruct((N, o_dim), jnp.float32),
                   jax.ShapeDtypeStruct((N, o_dim), jnp.float32)],
    )(h, c["Wl"], c["bl"][None, :], c["Wr"], c["br"][None, :])


def _alpha_body(xlg_ref, xrg_ref, w_ref, we_ref, att_ref, alpha_ref):
    ob = pl.program_id(1)
    z = xlg_ref[...] + xrg_ref[...] + _dot(w_ref[...], we_ref[...])
    part = jnp.sum(_leaky(z, 0.2) * att_ref[...], axis=-1, keepdims=True)

    @pl.when(ob == 0)
    def _():
        alpha_ref[...] = part

    @pl.when(ob != 0)
    def _():
        alpha_ref[...] += part


def _alpha(xlg, xrg, w, c, o_dim, be, bo):
    grid = (E // be, o_dim // bo)
    espec = pl.BlockSpec((be, bo), lambda i, j: (i, j))
    out = pl.pallas_call(
        _alpha_body,
        grid=grid,
        in_specs=[espec, espec,
                  pl.BlockSpec((be, L0), lambda i, j: (i, 0)),
                  pl.BlockSpec((L0, bo), lambda i, j: (0, j)),
                  pl.BlockSpec((1, bo), lambda i, j: (0, j))],
        out_specs=pl.BlockSpec((be, 1), lambda i, j: (i, 0)),
        out_shape=jax.ShapeDtypeStruct((E, 1), jnp.float32),
    )(xlg, xrg, w, c["We"], c["att"][None, :])
    return out[:, 0]


def _lnleaky_body(raw_ref, bias_ref, g_ref, b_ref, out_ref):
    out_ref[...] = _leaky(_ln(raw_ref[...] + bias_ref[...],
                              g_ref[...], b_ref[...]), 0.01)


def _lnleaky(raw, bias, g, b, o_dim, bn):
    return pl.pallas_call(
        _lnleaky_body,
        grid=(N // bn,),
        in_specs=[pl.BlockSpec((bn, o_dim), lambda i: (i, 0)),
                  pl.BlockSpec((1, o_dim), lambda i: (0, 0)),
                  pl.BlockSpec((1, o_dim), lambda i: (0, 0)),
                  pl.BlockSpec((1, o_dim), lambda i: (0, 0))],
        out_specs=pl.BlockSpec((bn, o_dim), lambda i: (i, 0)),
        out_shape=jax.ShapeDtypeStruct((N, o_dim), jnp.float32),
    )(raw, bias[None, :], g[None, :], b[None, :])


def _xfpool_body(x3_ref, x2_ref, sk23_ref, x1_ref, sk13_ref, b23_ref, b13_ref,
                 batch_ref, pooled_ref):
    nb = pl.program_id(1)
    xf = (x3_ref[...] + _dot(x2_ref[...], sk23_ref[...]) + b23_ref[...]
          + _dot(x1_ref[...], sk13_ref[...]) + b13_ref[...])
    bt = batch_ref[0]
    oh = (jax.lax.broadcasted_iota(jnp.int32, (NUM_GRAPHS, bt.shape[1]), 0)
          == bt).astype(jnp.float32)
    part = _dot(oh, xf)

    @pl.when(nb == 0)
    def _():
        pooled_ref[...] = part

    @pl.when(nb != 0)
    def _():
        pooled_ref[...] += part


def _xfpool(x3p, x2p, x1p, batch3d, p, bn, bo):
    grid = (L3 // bo, N // bn)
    return pl.pallas_call(
        _xfpool_body,
        grid=grid,
        in_specs=[pl.BlockSpec((bn, bo), lambda j, i: (i, j)),
                  pl.BlockSpec((bn, L2), lambda j, i: (i, 0)),
                  pl.BlockSpec((L2, bo), lambda j, i: (0, j)),
                  pl.BlockSpec((bn, L1), lambda j, i: (i, 0)),
                  pl.BlockSpec((L1, bo), lambda j, i: (0, j)),
                  pl.BlockSpec((1, bo), lambda j, i: (0, j)),
                  pl.BlockSpec((1, bo), lambda j, i: (0, j)),
                  pl.BlockSpec((1, 1, bn), lambda j, i: (i, 0, 0))],
        out_specs=pl.BlockSpec((NUM_GRAPHS, bo), lambda j, i: (0, j)),
        out_shape=jax.ShapeDtypeStruct((NUM_GRAPHS, L3), jnp.float32),
    )(x3p, x2p, p["skip23"]["W"], x1p, p["skip13"]["W"],
      p["skip23"]["b"][None, :], p["skip13"]["b"][None, :], batch3d)


def _head_body(pooled_ref, w1_ref, b1_ref, w2_ref, b2_ref, out_ref):
    f = jax.nn.relu(_dot(pooled_ref[...], w1_ref[...]) + b1_ref[...])
    out_ref[...] = _dot(f, w2_ref[...]) + b2_ref[...]


def _head(pooled, p):
    return pl.pallas_call(
        _head_body,
        out_shape=jax.ShapeDtypeStruct((NUM_GRAPHS, NUM_CLASSES), jnp.float32),
    )(pooled, p["fc1"]["W"], p["fc1"]["b"][None, :],
      p["fc2"]["W"], p["fc2"]["b"][None, :])


_CONV_CFG = {
    1: dict(i_dim=L0, o_dim=L1, bn=2000, bo=512, abe=2000, abo=512),
    2: dict(i_dim=L1, o_dim=L2, bn=1000, bo=512, abe=1000, abo=512),
    3: dict(i_dim=L2, o_dim=L3, bn=400, bo=512, abe=800, abo=512),
}


def kernel(x, edge_index, batch, params):
    p = params
    src = edge_index[0]
    dst = edge_index[1]

    feats, h = _node_encoder(x, p)
    fa = jnp.take(feats, src, axis=0)
    fb = jnp.take(feats, dst, axis=0)
    w = _edge_mlp(fa, fb, p)

    def conv(hk, c, cfg):
        xl, xr = _dense2(hk, c, cfg["i_dim"], cfg["o_dim"], cfg["bn"],
                         cfg["bo"])
        xlg = jnp.take(xl, src, axis=0)
        xrg = jnp.take(xr, dst, axis=0)
        alpha = _alpha(xlg, xrg, w, c, cfg["o_dim"], cfg["abe"], cfg["abo"])
        amax = jax.ops.segment_max(alpha, dst, num_segments=N)
        amax = jnp.where(jnp.isfinite(amax), amax, 0.0)
        ex = jnp.exp(alpha - amax[dst])
        den = jax.ops.segment_sum(ex, dst, num_segments=N)
        a = ex / (den[dst] + 1e-16)
        return jax.ops.segment_sum(xlg * a[:, None], dst, num_segments=N)

    r1 = conv(h, p["conv1"], _CONV_CFG[1])
    x1p = _lnleaky(r1, p["conv1"]["bias"], p["ln1"]["g"], p["ln1"]["b"],
                   L1, 2000)
    r2 = conv(x1p, p["conv2"], _CONV_CFG[2])
    x2p = _lnleaky(r2, p["conv2"]["bias"], p["ln2"]["g"], p["ln2"]["b"],
                   L2, 1000)
    r3 = conv(x2p, p["conv3"], _CONV_CFG[3])
    x3p = _lnleaky(r3, p["conv3"]["bias"], p["ln3"]["g"], p["ln3"]["b"],
                   L3, 400)

    batch3d = batch.reshape(N // 1000, 1, 1000)
    pooled = _xfpool(x3p, x2p, x1p, batch3d, p, 1000, 512)
    return _head(pooled, p)


# speedup vs baseline: 1.0217x; 1.0217x over previous
import functools

import jax
import jax.numpy as jnp
from jax.experimental import pallas as pl
from jax.experimental.pallas import tpu as pltpu

N = 10000
E = 160000
NUM_GRAPHS = 64
NUM_CLASSES = 40
EMBED = 32
L0 = 128
L1 = 512
L2 = 2048
L3 = 4096


def _leaky(v, s):
    return jnp.where(v >= 0, v, s * v)


def _ln(v, g, b):
    m = jnp.mean(v, axis=-1, keepdims=True)
    var = jnp.mean((v - m) ** 2, axis=-1, keepdims=True)
    return (v - m) / jnp.sqrt(var + 1e-5) * g + b


def _dot(a, b):
    return jnp.dot(a, b, preferred_element_type=jnp.float32)


def _dotb(a, b):
    return jnp.dot(a.astype(jnp.bfloat16), b.astype(jnp.bfloat16),
                   preferred_element_type=jnp.float32)


def _node_enc_body(x_ref, emby_ref, embx_ref, embp_ref, smw_ref, smb_ref,
                   how_ref, hob_ref, orw_ref, orb_ref, fmw_ref, fmb_ref,
                   g0_ref, b0_ref, feats_ref, h_ref):
    x = x_ref[...]

    def onehot(col, k):
        i = jax.lax.broadcasted_iota(jnp.int32, (1, k), 1)
        return (col.astype(jnp.int32) == i).astype(jnp.float32)

    cy = _dot(onehot(x[:, 0:1], 18), emby_ref[...])
    cx = _dot(onehot(x[:, 1:2], 11), embx_ref[...])
    coords = _leaky(_dot(jnp.concatenate([cy, cx], axis=1), smw_ref[...])
                    + smb_ref[...], 0.01)
    positions = _leaky(_dot(onehot(x[:, 2:3], 140), embp_ref[...]), 0.01)
    ht = x[:, 3:8]
    ht_e = (_dot(ht, how_ref[...]) + hob_ref[...]) / (
        jnp.sum(ht, axis=1, keepdims=True) + 1e-8)
    ori = x[:, 8:16]
    ori_e = (_dot(ori, orw_ref[...]) + orb_ref[...]) / (
        jnp.sum(ori, axis=1, keepdims=True) + 1e-8)
    feats = jnp.concatenate([coords, ori_e, ht_e, positions], axis=1)
    feats_ref[...] = feats
    h_ref[...] = jax.nn.relu(_ln(_dot(feats, fmw_ref[...]) + fmb_ref[...],
                                 g0_ref[...], b0_ref[...]))


def _node_encoder(x, p):
    outs = pl.pallas_call(
        _node_enc_body,
        out_shape=[jax.ShapeDtypeStruct((N, L0), jnp.float32),
                   jax.ShapeDtypeStruct((N, L0), jnp.float32)],
    )(x, p["emb_y"], p["emb_x"], p["emb_pos"],
      p["smoosh"]["W"], p["smoosh"]["b"][None, :],
      p["hold"]["W"], p["hold"]["b"][None, :],
      p["orient"]["W"], p["orient"]["b"][None, :],
      p["feat_mix"]["W"], p["feat_mix"]["b"][None, :],
      p["ln0"]["g"][None, :], p["ln0"]["b"][None, :])
    return outs[0], outs[1]


def _edge_mlp_body(fa_ref, fb_ref, w1a_ref, w1b_ref, b1_ref, g1_ref, b1n_ref,
                   w2_ref, b2_ref, g0_ref, b0n_ref, w_ref):
    u = (_dotb(fa_ref[...][:, :96], w1a_ref[...])
         + _dotb(fb_ref[...][:, :96], w1b_ref[...]) + b1_ref[...])
    u = jax.nn.relu(_ln(u, g1_ref[...], b1n_ref[...]))
    v = _dotb(u, w2_ref[...]) + b2_ref[...]
    w_ref[...] = _leaky(_ln(v, g0_ref[...], b0n_ref[...]), 0.01)


def _edge_mlp(fa, fb, p, be=2000):
    grid = (E // be,)
    espec = pl.BlockSpec((be, L0), lambda i: (i, 0))
    wspec = lambda shape: pl.BlockSpec(shape, lambda i: (0, 0))
    w1 = p["ew1"]["W"]
    return pl.pallas_call(
        _edge_mlp_body,
        grid=grid,
        in_specs=[espec, espec,
                  wspec((96, L1)), wspec((96, L1)), wspec((1, L1)),
                  wspec((1, L1)), wspec((1, L1)),
                  wspec((L1, L0)), wspec((1, L0)),
                  wspec((1, L0)), wspec((1, L0))],
        out_specs=espec,
        out_shape=jax.ShapeDtypeStruct((E, L0), jnp.float32),
    )(fa, fb, w1[:96], w1[96:], p["ew1"]["b"][None, :],
      p["ln1"]["g"][None, :], p["ln1"]["b"][None, :],
      p["ew2"]["W"], p["ew2"]["b"][None, :],
      p["ln0"]["g"][None, :], p["ln0"]["b"][None, :])


def _dense2_body(h_ref, wl_ref, bl_ref, wr_ref, br_ref, xl_ref, xr_ref):
    h = h_ref[...].astype(jnp.bfloat16)
    xl_ref[...] = (_dotb(h, wl_ref[...]) + bl_ref[...]).astype(jnp.bfloat16)
    xr_ref[...] = (_dotb(h, wr_ref[...]) + br_ref[...]).astype(jnp.bfloat16)


def _dense2(h, c, i_dim, o_dim, bn, bo):
    grid = (N // bn, o_dim // bo)
    ospec = pl.BlockSpec((bn, bo), lambda i, j: (i, j))
    return pl.pallas_call(
        _dense2_body,
        grid=grid,
        in_specs=[pl.BlockSpec((bn, i_dim), lambda i, j: (i, 0)),
                  pl.BlockSpec((i_dim, bo), lambda i, j: (0, j)),
                  pl.BlockSpec((1, bo), lambda i, j: (0, j)),
                  pl.BlockSpec((i_dim, bo), lambda i, j: (0, j)),
                  pl.BlockSpec((1, bo), lambda i, j: (0, j))],
        out_specs=[ospec, ospec],
        out_shape=[jax.ShapeDtypeStruct((N, o_dim), jnp.bfloat16),
                   jax.ShapeDtypeStruct((N, o_dim), jnp.bfloat16)],
    )(h, c["Wl"], c["bl"][None, :], c["Wr"], c["br"][None, :])


def _alpha_body(xlg_ref, xrg_ref, w_ref, we_ref, att_ref, alpha_ref):
    ob = pl.program_id(1)
    z = (xlg_ref[...].astype(jnp.float32) + xrg_ref[...].astype(jnp.float32)
         + _dotb(w_ref[...], we_ref[...]))
    part = jnp.sum(_leaky(z, 0.2) * att_ref[...], axis=-1, keepdims=True)

    @pl.when(ob == 0)
    def _():
        alpha_ref[...] = part

    @pl.when(ob != 0)
    def _():
        alpha_ref[...] += part


def _alpha(xlg, xrg, w, c, o_dim, be, bo):
    grid = (E // be, o_dim // bo)
    espec = pl.BlockSpec((be, bo), lambda i, j: (i, j))
    out = pl.pallas_call(
        _alpha_body,
        grid=grid,
        in_specs=[espec, espec,
                  pl.BlockSpec((be, L0), lambda i, j: (i, 0)),
                  pl.BlockSpec((L0, bo), lambda i, j: (0, j)),
                  pl.BlockSpec((1, bo), lambda i, j: (0, j))],
        out_specs=pl.BlockSpec((be, 1), lambda i, j: (i, 0)),
        out_shape=jax.ShapeDtypeStruct((E, 1), jnp.float32),
    )(xlg, xrg, w, c["We"], c["att"][None, :])
    return out[:, 0]


def _lnleaky_body(raw_ref, bias_ref, g_ref, b_ref, out_ref):
    out_ref[...] = _leaky(_ln(raw_ref[...] + bias_ref[...],
                              g_ref[...], b_ref[...]), 0.01)


def _lnleaky(raw, bias, g, b, o_dim, bn):
    return pl.pallas_call(
        _lnleaky_body,
        grid=(N // bn,),
        in_specs=[pl.BlockSpec((bn, o_dim), lambda i: (i, 0)),
                  pl.BlockSpec((1, o_dim), lambda i: (0, 0)),
                  pl.BlockSpec((1, o_dim), lambda i: (0, 0)),
                  pl.BlockSpec((1, o_dim), lambda i: (0, 0))],
        out_specs=pl.BlockSpec((bn, o_dim), lambda i: (i, 0)),
        out_shape=jax.ShapeDtypeStruct((N, o_dim), jnp.float32),
    )(raw, bias[None, :], g[None, :], b[None, :])


def _xfpool_body(x3_ref, x2_ref, sk23_ref, x1_ref, sk13_ref, b23_ref, b13_ref,
                 batch_ref, pooled_ref):
    nb = pl.program_id(1)
    xf = (x3_ref[...] + _dotb(x2_ref[...], sk23_ref[...]) + b23_ref[...]
          + _dotb(x1_ref[...], sk13_ref[...]) + b13_ref[...])
    bt = batch_ref[0]
    oh = (jax.lax.broadcasted_iota(jnp.int32, (NUM_GRAPHS, bt.shape[1]), 0)
          == bt).astype(jnp.bfloat16)
    part = _dotb(oh, xf)

    @pl.when(nb == 0)
    def _():
        pooled_ref[...] = part

    @pl.when(nb != 0)
    def _():
        pooled_ref[...] += part


def _xfpool(x3p, x2p, x1p, batch3d, p, bn, bo):
    grid = (L3 // bo, N // bn)
    return pl.pallas_call(
        _xfpool_body,
        grid=grid,
        in_specs=[pl.BlockSpec((bn, bo), lambda j, i: (i, j)),
                  pl.BlockSpec((bn, L2), lambda j, i: (i, 0)),
                  pl.BlockSpec((L2, bo), lambda j, i: (0, j)),
                  pl.BlockSpec((bn, L1), lambda j, i: (i, 0)),
                  pl.BlockSpec((L1, bo), lambda j, i: (0, j)),
                  pl.BlockSpec((1, bo), lambda j, i: (0, j)),
                  pl.BlockSpec((1, bo), lambda j, i: (0, j)),
                  pl.BlockSpec((1, 1, bn), lambda j, i: (i, 0, 0))],
        out_specs=pl.BlockSpec((NUM_GRAPHS, bo), lambda j, i: (0, j)),
        out_shape=jax.ShapeDtypeStruct((NUM_GRAPHS, L3), jnp.float32),
    )(x3p, x2p, p["skip23"]["W"], x1p, p["skip13"]["W"],
      p["skip23"]["b"][None, :], p["skip13"]["b"][None, :], batch3d)


def _head_body(pooled_ref, w1_ref, b1_ref, w2_ref, b2_ref, out_ref):
    f = jax.nn.relu(_dot(pooled_ref[...], w1_ref[...]) + b1_ref[...])
    out_ref[...] = _dot(f, w2_ref[...]) + b2_ref[...]


def _head(pooled, p):
    return pl.pallas_call(
        _head_body,
        out_shape=jax.ShapeDtypeStruct((NUM_GRAPHS, NUM_CLASSES), jnp.float32),
    )(pooled, p["fc1"]["W"], p["fc1"]["b"][None, :],
      p["fc2"]["W"], p["fc2"]["b"][None, :])


_CONV_CFG = {
    1: dict(i_dim=L0, o_dim=L1, bn=2000, bo=512, abe=2000, abo=512),
    2: dict(i_dim=L1, o_dim=L2, bn=400, bo=512, abe=800, abo=512),
    3: dict(i_dim=L2, o_dim=L3, bn=400, bo=512, abe=800, abo=512),
}


def kernel(x, edge_index, batch, params):
    p = params
    src = edge_index[0]
    dst = edge_index[1]

    feats, h = _node_encoder(x, p)
    fa = jnp.take(feats, src, axis=0)
    fb = jnp.take(feats, dst, axis=0)
    w = _edge_mlp(fa, fb, p)

    def conv(hk, c, cfg):
        xl, xr = _dense2(hk, c, cfg["i_dim"], cfg["o_dim"], cfg["bn"],
                         cfg["bo"])
        xlg = jnp.take(xl, src, axis=0)
        xrg = jnp.take(xr, dst, axis=0)
        alpha = _alpha(xlg, xrg, w, c, cfg["o_dim"], cfg["abe"], cfg["abo"])
        amax = jax.ops.segment_max(alpha, dst, num_segments=N)
        amax = jnp.where(jnp.isfinite(amax), amax, 0.0)
        ex = jnp.exp(alpha - amax[dst])
        den = jax.ops.segment_sum(ex, dst, num_segments=N)
        a = ex / (den[dst] + 1e-16)
        return jax.ops.segment_sum(xlg * a[:, None], dst, num_segments=N)

    r1 = conv(h, p["conv1"], _CONV_CFG[1])
    x1p = _lnleaky(r1, p["conv1"]["bias"], p["ln1"]["g"], p["ln1"]["b"],
                   L1, 2000)
    r2 = conv(x1p, p["conv2"], _CONV_CFG[2])
    x2p = _lnleaky(r2, p["conv2"]["bias"], p["ln2"]["g"], p["ln2"]["b"],
                   L2, 1000)
    r3 = conv(x2p, p["conv3"], _CONV_CFG[3])
    x3p = _lnleaky(r3, p["conv3"]["bias"], p["ln3"]["g"], p["ln3"]["b"],
                   L3, 400)

    batch3d = batch.reshape(N // 1000, 1, 1000)
    pooled = _xfpool(x3p, x2p, x1p, batch3d, p, 1000, 512)
    return _head(pooled, p)


# speedup vs baseline: 1.5119x; 1.4798x over previous
import functools

import jax
import jax.numpy as jnp
from jax.experimental import pallas as pl
from jax.experimental.pallas import tpu as pltpu

N = 10000
E = 160000
NUM_GRAPHS = 64
NUM_CLASSES = 40
EMBED = 32
L0 = 128
L1 = 512
L2 = 2048
L3 = 4096
NP = 10240


def _leaky(v, s):
    return jnp.where(v >= 0, v, s * v)


def _ln(v, g, b):
    m = jnp.mean(v, axis=-1, keepdims=True)
    var = jnp.mean((v - m) ** 2, axis=-1, keepdims=True)
    return (v - m) / jnp.sqrt(var + 1e-5) * g + b


def _dot(a, b):
    return jnp.dot(a, b, preferred_element_type=jnp.float32)


def _dotb(a, b):
    return jnp.dot(a.astype(jnp.bfloat16), b.astype(jnp.bfloat16),
                   preferred_element_type=jnp.float32)


def _node_enc_body(x_ref, emby_ref, embx_ref, embp_ref, smw_ref, smb_ref,
                   how_ref, hob_ref, orw_ref, orb_ref, fmw_ref, fmb_ref,
                   g0_ref, b0_ref, feats_ref, h_ref):
    x = x_ref[...]

    def onehot(col, k):
        i = jax.lax.broadcasted_iota(jnp.int32, (1, k), 1)
        return (col.astype(jnp.int32) == i).astype(jnp.float32)

    cy = _dot(onehot(x[:, 0:1], 18), emby_ref[...])
    cx = _dot(onehot(x[:, 1:2], 11), embx_ref[...])
    coords = _leaky(_dot(jnp.concatenate([cy, cx], axis=1), smw_ref[...])
                    + smb_ref[...], 0.01)
    positions = _leaky(_dot(onehot(x[:, 2:3], 140), embp_ref[...]), 0.01)
    ht = x[:, 3:8]
    ht_e = (_dot(ht, how_ref[...]) + hob_ref[...]) / (
        jnp.sum(ht, axis=1, keepdims=True) + 1e-8)
    ori = x[:, 8:16]
    ori_e = (_dot(ori, orw_ref[...]) + orb_ref[...]) / (
        jnp.sum(ori, axis=1, keepdims=True) + 1e-8)
    feats = jnp.concatenate([coords, ori_e, ht_e, positions], axis=1)
    feats_ref[...] = feats
    h_ref[...] = jax.nn.relu(_ln(_dot(feats, fmw_ref[...]) + fmb_ref[...],
                                 g0_ref[...], b0_ref[...]))


def _node_encoder(x, p):
    outs = pl.pallas_call(
        _node_enc_body,
        out_shape=[jax.ShapeDtypeStruct((N, L0), jnp.float32),
                   jax.ShapeDtypeStruct((N, L0), jnp.float32)],
    )(x, p["emb_y"], p["emb_x"], p["emb_pos"],
      p["smoosh"]["W"], p["smoosh"]["b"][None, :],
      p["hold"]["W"], p["hold"]["b"][None, :],
      p["orient"]["W"], p["orient"]["b"][None, :],
      p["feat_mix"]["W"], p["feat_mix"]["b"][None, :],
      p["ln0"]["g"][None, :], p["ln0"]["b"][None, :])
    return outs[0], outs[1]


def _edge_mlp_body(fa_ref, fb_ref, w1a_ref, w1b_ref, b1_ref, g1_ref, b1n_ref,
                   w2_ref, b2_ref, g0_ref, b0n_ref, w_ref):
    u = (_dotb(fa_ref[...][:, :96], w1a_ref[...])
         + _dotb(fb_ref[...][:, :96], w1b_ref[...]) + b1_ref[...])
    u = jax.nn.relu(_ln(u, g1_ref[...], b1n_ref[...]))
    v = _dotb(u, w2_ref[...]) + b2_ref[...]
    w_ref[...] = _leaky(_ln(v, g0_ref[...], b0n_ref[...]), 0.01)


def _edge_mlp(fa, fb, p, be=2000):
    grid = (E // be,)
    espec = pl.BlockSpec((be, L0), lambda i: (i, 0))
    wspec = lambda shape: pl.BlockSpec(shape, lambda i: (0, 0))
    w1 = p["ew1"]["W"]
    return pl.pallas_call(
        _edge_mlp_body,
        grid=grid,
        in_specs=[espec, espec,
                  wspec((96, L1)), wspec((96, L1)), wspec((1, L1)),
                  wspec((1, L1)), wspec((1, L1)),
                  wspec((L1, L0)), wspec((1, L0)),
                  wspec((1, L0)), wspec((1, L0))],
        out_specs=espec,
        out_shape=jax.ShapeDtypeStruct((E, L0), jnp.float32),
    )(fa, fb, w1[:96], w1[96:], p["ew1"]["b"][None, :],
      p["ln1"]["g"][None, :], p["ln1"]["b"][None, :],
      p["ew2"]["W"], p["ew2"]["b"][None, :],
      p["ln0"]["g"][None, :], p["ln0"]["b"][None, :])


def _dense2_body(h_ref, wl_ref, bl_ref, wr_ref, br_ref, xl_ref, xr_ref):
    h = h_ref[...].astype(jnp.bfloat16)
    xl_ref[...] = (_dotb(h, wl_ref[...]) + bl_ref[...]).astype(jnp.bfloat16)
    xr_ref[...] = (_dotb(h, wr_ref[...]) + br_ref[...]).astype(jnp.bfloat16)


def _dense2(h, c, i_dim, o_dim, bn, bo):
    grid = (N // bn, o_dim // bo)
    ospec = pl.BlockSpec((bn, bo), lambda i, j: (i, j))
    return pl.pallas_call(
        _dense2_body,
        grid=grid,
        in_specs=[pl.BlockSpec((bn, i_dim), lambda i, j: (i, 0)),
                  pl.BlockSpec((i_dim, bo), lambda i, j: (0, j)),
                  pl.BlockSpec((1, bo), lambda i, j: (0, j)),
                  pl.BlockSpec((i_dim, bo), lambda i, j: (0, j)),
                  pl.BlockSpec((1, bo), lambda i, j: (0, j))],
        out_specs=[ospec, ospec],
        out_shape=[jax.ShapeDtypeStruct((N, o_dim), jnp.bfloat16),
                   jax.ShapeDtypeStruct((N, o_dim), jnp.bfloat16)],
    )(h, c["Wl"], c["bl"][None, :], c["Wr"], c["br"][None, :])


def _alpha_body(xlg_ref, xrg_ref, w_ref, we_ref, att_ref, alpha_ref):
    ob = pl.program_id(1)
    z = (xlg_ref[...].astype(jnp.float32) + xrg_ref[...].astype(jnp.float32)
         + _dotb(w_ref[...], we_ref[...]))
    part = jnp.sum(_leaky(z, 0.2) * att_ref[...], axis=-1, keepdims=True)

    @pl.when(ob == 0)
    def _():
        alpha_ref[...] = part

    @pl.when(ob != 0)
    def _():
        alpha_ref[...] += part


def _alpha(xlg, xrg, w, c, o_dim, be, bo):
    grid = (E // be, o_dim // bo)
    espec = pl.BlockSpec((be, bo), lambda i, j: (i, j))
    out = pl.pallas_call(
        _alpha_body,
        grid=grid,
        in_specs=[espec, espec,
                  pl.BlockSpec((be, L0), lambda i, j: (i, 0)),
                  pl.BlockSpec((L0, bo), lambda i, j: (0, j)),
                  pl.BlockSpec((1, bo), lambda i, j: (0, j))],
        out_specs=pl.BlockSpec((be, 1), lambda i, j: (i, 0)),
        out_shape=jax.ShapeDtypeStruct((E, 1), jnp.float32),
    )(xlg, xrg, w, c["We"], c["att"][None, :])
    return out[:, 0]


def _aggmm_body(a_ref, xl_ref, out_ref, den_ref):
    k = pl.program_id(1)
    nk = pl.num_programs(1)
    ab = a_ref[...]
    part = _dotb(ab, xl_ref[...])
    dpart = jnp.sum(ab, axis=1, keepdims=True)

    @pl.when(k == 0)
    def _():
        out_ref[...] = part
        den_ref[...] = dpart

    @pl.when(k != 0)
    def _():
        out_ref[...] += part
        den_ref[...] += dpart

    @pl.when(k == nk - 1)
    def _():
        out_ref[...] = out_ref[...] / (den_ref[...] + 1e-16)


def _aggmm(a_mat, xl, o_dim, bn, bk):
    grid = (NP // bn, NP // bk)
    return pl.pallas_call(
        _aggmm_body,
        grid=grid,
        in_specs=[pl.BlockSpec((bn, bk), lambda i, k: (i, k)),
                  pl.BlockSpec((bk, o_dim), lambda i, k: (k, 0))],
        out_specs=pl.BlockSpec((bn, o_dim), lambda i, k: (i, 0)),
        out_shape=jax.ShapeDtypeStruct((NP, o_dim), jnp.float32),
        scratch_shapes=[pltpu.VMEM((bn, 1), jnp.float32)],
    )(a_mat, xl)


def _lnleaky_body(raw_ref, bias_ref, g_ref, b_ref, out_ref):
    out_ref[...] = _leaky(_ln(raw_ref[...] + bias_ref[...],
                              g_ref[...], b_ref[...]), 0.01)


def _lnleaky(raw, bias, g, b, o_dim, bn):
    return pl.pallas_call(
        _lnleaky_body,
        grid=(N // bn,),
        in_specs=[pl.BlockSpec((bn, o_dim), lambda i: (i, 0)),
                  pl.BlockSpec((1, o_dim), lambda i: (0, 0)),
                  pl.BlockSpec((1, o_dim), lambda i: (0, 0)),
                  pl.BlockSpec((1, o_dim), lambda i: (0, 0))],
        out_specs=pl.BlockSpec((bn, o_dim), lambda i: (i, 0)),
        out_shape=jax.ShapeDtypeStruct((N, o_dim), jnp.float32),
    )(raw, bias[None, :], g[None, :], b[None, :])


def _xfpool_body(x3_ref, x2_ref, sk23_ref, x1_ref, sk13_ref, b23_ref, b13_ref,
                 batch_ref, pooled_ref):
    nb = pl.program_id(1)
    xf = (x3_ref[...] + _dotb(x2_ref[...], sk23_ref[...]) + b23_ref[...]
          + _dotb(x1_ref[...], sk13_ref[...]) + b13_ref[...])
    bt = batch_ref[0]
    oh = (jax.lax.broadcasted_iota(jnp.int32, (NUM_GRAPHS, bt.shape[1]), 0)
          == bt).astype(jnp.bfloat16)
    part = _dotb(oh, xf)

    @pl.when(nb == 0)
    def _():
        pooled_ref[...] = part

    @pl.when(nb != 0)
    def _():
        pooled_ref[...] += part


def _xfpool(x3p, x2p, x1p, batch3d, p, bn, bo):
    grid = (L3 // bo, N // bn)
    return pl.pallas_call(
        _xfpool_body,
        grid=grid,
        in_specs=[pl.BlockSpec((bn, bo), lambda j, i: (i, j)),
                  pl.BlockSpec((bn, L2), lambda j, i: (i, 0)),
                  pl.BlockSpec((L2, bo), lambda j, i: (0, j)),
                  pl.BlockSpec((bn, L1), lambda j, i: (i, 0)),
                  pl.BlockSpec((L1, bo), lambda j, i: (0, j)),
                  pl.BlockSpec((1, bo), lambda j, i: (0, j)),
                  pl.BlockSpec((1, bo), lambda j, i: (0, j)),
                  pl.BlockSpec((1, 1, bn), lambda j, i: (i, 0, 0))],
        out_specs=pl.BlockSpec((NUM_GRAPHS, bo), lambda j, i: (0, j)),
        out_shape=jax.ShapeDtypeStruct((NUM_GRAPHS, L3), jnp.float32),
    )(x3p, x2p, p["skip23"]["W"], x1p, p["skip13"]["W"],
      p["skip23"]["b"][None, :], p["skip13"]["b"][None, :], batch3d)


def _head_body(pooled_ref, w1_ref, b1_ref, w2_ref, b2_ref, out_ref):
    f = jax.nn.relu(_dot(pooled_ref[...], w1_ref[...]) + b1_ref[...])
    out_ref[...] = _dot(f, w2_ref[...]) + b2_ref[...]


def _head(pooled, p):
    return pl.pallas_call(
        _head_body,
        out_shape=jax.ShapeDtypeStruct((NUM_GRAPHS, NUM_CLASSES), jnp.float32),
    )(pooled, p["fc1"]["W"], p["fc1"]["b"][None, :],
      p["fc2"]["W"], p["fc2"]["b"][None, :])


_CONV_CFG = {
    1: dict(i_dim=L0, o_dim=L1, bn=2000, bo=512, abe=2000, abo=512,
            mbn=512, mbk=2048),
    2: dict(i_dim=L1, o_dim=L2, bn=400, bo=512, abe=800, abo=512,
            mbn=512, mbk=2048),
    3: dict(i_dim=L2, o_dim=L3, bn=400, bo=512, abe=800, abo=512,
            mbn=512, mbk=1024),
}


def kernel(x, edge_index, batch, params):
    p = params
    src = edge_index[0]
    dst = edge_index[1]

    feats, h = _node_encoder(x, p)
    fa = jnp.take(feats, src, axis=0)
    fb = jnp.take(feats, dst, axis=0)
    w = _edge_mlp(fa, fb, p)

    def conv(hk, c, cfg):
        xl, xr = _dense2(hk, c, cfg["i_dim"], cfg["o_dim"], cfg["bn"],
                         cfg["bo"])
        xlg = jnp.take(xl, src, axis=0)
        xrg = jnp.take(xr, dst, axis=0)
        alpha = _alpha(xlg, xrg, w, c, cfg["o_dim"], cfg["abe"], cfg["abo"])
        amax = jax.ops.segment_max(alpha, dst, num_segments=N)
        amax = jnp.where(jnp.isfinite(amax), amax, 0.0)
        ex = jnp.exp(alpha - amax[dst])
        lin = dst * NP + src
        a_mat = jnp.zeros((NP * NP,), jnp.float32).at[lin].add(ex).reshape(NP, NP)
        xlp = jnp.pad(xl, ((0, NP - N), (0, 0)))
        return _aggmm(a_mat, xlp, cfg["o_dim"], cfg["mbn"], cfg["mbk"])[:N]

    r1 = conv(h, p["conv1"], _CONV_CFG[1])
    x1p = _lnleaky(r1, p["conv1"]["bias"], p["ln1"]["g"], p["ln1"]["b"],
                   L1, 2000)
    r2 = conv(x1p, p["conv2"], _CONV_CFG[2])
    x2p = _lnleaky(r2, p["conv2"]["bias"], p["ln2"]["g"], p["ln2"]["b"],
                   L2, 1000)
    r3 = conv(x2p, p["conv3"], _CONV_CFG[3])
    x3p = _lnleaky(r3, p["conv3"]["bias"], p["ln3"]["g"], p["ln3"]["b"],
                   L3, 400)

    batch3d = batch.reshape(N // 1000, 1, 1000)
    pooled = _xfpool(x3p, x2p, x1p, batch3d, p, 1000, 512)
    return _head(pooled, p)


# speedup vs baseline: 1.5214x; 1.0063x over previous
import functools

import jax
import jax.numpy as jnp
from jax.experimental import pallas as pl
from jax.experimental.pallas import tpu as pltpu

N = 10000
E = 160000
NUM_GRAPHS = 64
NUM_CLASSES = 40
EMBED = 32
L0 = 128
L1 = 512
L2 = 2048
L3 = 4096
NP = 10240


def _leaky(v, s):
    return jnp.where(v >= 0, v, s * v)


def _ln(v, g, b):
    m = jnp.mean(v, axis=-1, keepdims=True)
    var = jnp.mean((v - m) ** 2, axis=-1, keepdims=True)
    return (v - m) / jnp.sqrt(var + 1e-5) * g + b


def _dot(a, b):
    return jnp.dot(a, b, preferred_element_type=jnp.float32)


def _dotb(a, b):
    return jnp.dot(a.astype(jnp.bfloat16), b.astype(jnp.bfloat16),
                   preferred_element_type=jnp.float32)


def _node_enc_body(x_ref, emby_ref, embx_ref, embp_ref, smw_ref, smb_ref,
                   how_ref, hob_ref, orw_ref, orb_ref, fmw_ref, fmb_ref,
                   g0_ref, b0_ref, feats_ref, h_ref):
    x = x_ref[...]

    def onehot(col, k):
        i = jax.lax.broadcasted_iota(jnp.int32, (1, k), 1)
        return (col.astype(jnp.int32) == i).astype(jnp.float32)

    cy = _dot(onehot(x[:, 0:1], 18), emby_ref[...])
    cx = _dot(onehot(x[:, 1:2], 11), embx_ref[...])
    coords = _leaky(_dot(jnp.concatenate([cy, cx], axis=1), smw_ref[...])
                    + smb_ref[...], 0.01)
    positions = _leaky(_dot(onehot(x[:, 2:3], 140), embp_ref[...]), 0.01)
    ht = x[:, 3:8]
    ht_e = (_dot(ht, how_ref[...]) + hob_ref[...]) / (
        jnp.sum(ht, axis=1, keepdims=True) + 1e-8)
    ori = x[:, 8:16]
    ori_e = (_dot(ori, orw_ref[...]) + orb_ref[...]) / (
        jnp.sum(ori, axis=1, keepdims=True) + 1e-8)
    feats = jnp.concatenate([coords, ori_e, ht_e, positions], axis=1)
    feats_ref[...] = feats
    h_ref[...] = jax.nn.relu(_ln(_dot(feats, fmw_ref[...]) + fmb_ref[...],
                                 g0_ref[...], b0_ref[...]))


def _node_encoder(x, p):
    outs = pl.pallas_call(
        _node_enc_body,
        out_shape=[jax.ShapeDtypeStruct((N, L0), jnp.float32),
                   jax.ShapeDtypeStruct((N, L0), jnp.float32)],
    )(x, p["emb_y"], p["emb_x"], p["emb_pos"],
      p["smoosh"]["W"], p["smoosh"]["b"][None, :],
      p["hold"]["W"], p["hold"]["b"][None, :],
      p["orient"]["W"], p["orient"]["b"][None, :],
      p["feat_mix"]["W"], p["feat_mix"]["b"][None, :],
      p["ln0"]["g"][None, :], p["ln0"]["b"][None, :])
    return outs[0], outs[1]


def _edge_mlp_body(fa_ref, fb_ref, w1a_ref, w1b_ref, b1_ref, g1_ref, b1n_ref,
                   w2_ref, b2_ref, g0_ref, b0n_ref, w_ref):
    u = (_dotb(fa_ref[...][:, :96], w1a_ref[...])
         + _dotb(fb_ref[...][:, :96], w1b_ref[...]) + b1_ref[...])
    u = jax.nn.relu(_ln(u, g1_ref[...], b1n_ref[...]))
    v = _dotb(u, w2_ref[...]) + b2_ref[...]
    w_ref[...] = _leaky(_ln(v, g0_ref[...], b0n_ref[...]), 0.01)


def _edge_mlp(fa, fb, p, be=2000):
    grid = (E // be,)
    espec = pl.BlockSpec((be, L0), lambda i: (i, 0))
    wspec = lambda shape: pl.BlockSpec(shape, lambda i: (0, 0))
    w1 = p["ew1"]["W"]
    return pl.pallas_call(
        _edge_mlp_body,
        grid=grid,
        in_specs=[espec, espec,
                  wspec((96, L1)), wspec((96, L1)), wspec((1, L1)),
                  wspec((1, L1)), wspec((1, L1)),
                  wspec((L1, L0)), wspec((1, L0)),
                  wspec((1, L0)), wspec((1, L0))],
        out_specs=espec,
        out_shape=jax.ShapeDtypeStruct((E, L0), jnp.float32),
    )(fa, fb, w1[:96], w1[96:], p["ew1"]["b"][None, :],
      p["ln1"]["g"][None, :], p["ln1"]["b"][None, :],
      p["ew2"]["W"], p["ew2"]["b"][None, :],
      p["ln0"]["g"][None, :], p["ln0"]["b"][None, :])


def _dense2_body(h_ref, wl_ref, bl_ref, wr_ref, br_ref, xl_ref, xr_ref):
    h = h_ref[...].astype(jnp.bfloat16)
    xl_ref[...] = (_dotb(h, wl_ref[...]) + bl_ref[...]).astype(jnp.bfloat16)
    xr_ref[...] = (_dotb(h, wr_ref[...]) + br_ref[...]).astype(jnp.bfloat16)


def _dense2(h, c, i_dim, o_dim, bn, bo):
    grid = (N // bn, o_dim // bo)
    ospec = pl.BlockSpec((bn, bo), lambda i, j: (i, j))
    return pl.pallas_call(
        _dense2_body,
        grid=grid,
        in_specs=[pl.BlockSpec((bn, i_dim), lambda i, j: (i, 0)),
                  pl.BlockSpec((i_dim, bo), lambda i, j: (0, j)),
                  pl.BlockSpec((1, bo), lambda i, j: (0, j)),
                  pl.BlockSpec((i_dim, bo), lambda i, j: (0, j)),
                  pl.BlockSpec((1, bo), lambda i, j: (0, j))],
        out_specs=[ospec, ospec],
        out_shape=[jax.ShapeDtypeStruct((N, o_dim), jnp.bfloat16),
                   jax.ShapeDtypeStruct((N, o_dim), jnp.bfloat16)],
    )(h, c["Wl"], c["bl"][None, :], c["Wr"], c["br"][None, :])


def _alpha_body(xlg_ref, xrg_ref, w_ref, we_ref, att_ref, alpha_ref):
    ob = pl.program_id(1)
    z = (xlg_ref[...].astype(jnp.float32) + xrg_ref[...].astype(jnp.float32)
         + _dotb(w_ref[...], we_ref[...]))
    part = jnp.sum(_leaky(z, 0.2) * att_ref[...], axis=-1, keepdims=True)

    @pl.when(ob == 0)
    def _():
        alpha_ref[...] = part

    @pl.when(ob != 0)
    def _():
        alpha_ref[...] += part


def _alpha(xlg, xrg, w, c, o_dim, be, bo):
    ne = xlg.shape[0]
    grid = (ne // be, o_dim // bo)
    espec = pl.BlockSpec((be, bo), lambda i, j: (i, j))
    out = pl.pallas_call(
        _alpha_body,
        grid=grid,
        in_specs=[espec, espec,
                  pl.BlockSpec((be, L0), lambda i, j: (i, 0)),
                  pl.BlockSpec((L0, bo), lambda i, j: (0, j)),
                  pl.BlockSpec((1, bo), lambda i, j: (0, j))],
        out_specs=pl.BlockSpec((be, 1), lambda i, j: (i, 0)),
        out_shape=jax.ShapeDtypeStruct((ne, 1), jnp.float32),
    )(xlg, xrg, w, c["We"], c["att"][None, :])
    return out[:, 0]


def _aggmm_body(a_ref, xl_ref, out_ref, den_ref):
    k = pl.program_id(1)
    nk = pl.num_programs(1)
    ab = a_ref[...]
    part = _dotb(ab, xl_ref[...])
    dpart = jnp.sum(ab, axis=1, keepdims=True)

    @pl.when(k == 0)
    def _():
        out_ref[...] = part
        den_ref[...] = dpart

    @pl.when(k != 0)
    def _():
        out_ref[...] += part
        den_ref[...] += dpart

    @pl.when(k == nk - 1)
    def _():
        out_ref[...] = out_ref[...] / (den_ref[...] + 1e-16)


def _aggmm(a_mat, xl, o_dim, bn, bk):
    grid = (NP // bn, NP // bk)
    return pl.pallas_call(
        _aggmm_body,
        grid=grid,
        in_specs=[pl.BlockSpec((bn, bk), lambda i, k: (i, k)),
                  pl.BlockSpec((bk, o_dim), lambda i, k: (k, 0))],
        out_specs=pl.BlockSpec((bn, o_dim), lambda i, k: (i, 0)),
        out_shape=jax.ShapeDtypeStruct((NP, o_dim), jnp.float32),
        scratch_shapes=[pltpu.VMEM((bn, 1), jnp.float32)],
    )(a_mat, xl)


def _lnleaky_body(raw_ref, bias_ref, g_ref, b_ref, out_ref):
    out_ref[...] = _leaky(_ln(raw_ref[...] + bias_ref[...],
                              g_ref[...], b_ref[...]), 0.01)


def _lnleaky(raw, bias, g, b, o_dim, bn):
    return pl.pallas_call(
        _lnleaky_body,
        grid=(N // bn,),
        in_specs=[pl.BlockSpec((bn, o_dim), lambda i: (i, 0)),
                  pl.BlockSpec((1, o_dim), lambda i: (0, 0)),
                  pl.BlockSpec((1, o_dim), lambda i: (0, 0)),
                  pl.BlockSpec((1, o_dim), lambda i: (0, 0))],
        out_specs=pl.BlockSpec((bn, o_dim), lambda i: (i, 0)),
        out_shape=jax.ShapeDtypeStruct((N, o_dim), jnp.float32),
    )(raw, bias[None, :], g[None, :], b[None, :])


def _xfpool_body(x3_ref, x2_ref, sk23_ref, x1_ref, sk13_ref, b23_ref, b13_ref,
                 batch_ref, pooled_ref):
    nb = pl.program_id(1)
    xf = (x3_ref[...] + _dotb(x2_ref[...], sk23_ref[...]) + b23_ref[...]
          + _dotb(x1_ref[...], sk13_ref[...]) + b13_ref[...])
    bt = batch_ref[0]
    oh = (jax.lax.broadcasted_iota(jnp.int32, (NUM_GRAPHS, bt.shape[1]), 0)
          == bt).astype(jnp.bfloat16)
    part = _dotb(oh, xf)

    @pl.when(nb == 0)
    def _():
        pooled_ref[...] = part

    @pl.when(nb != 0)
    def _():
        pooled_ref[...] += part


def _xfpool(x3p, x2p, x1p, batch3d, p, bn, bo):
    grid = (L3 // bo, N // bn)
    return pl.pallas_call(
        _xfpool_body,
        grid=grid,
        in_specs=[pl.BlockSpec((bn, bo), lambda j, i: (i, j)),
                  pl.BlockSpec((bn, L2), lambda j, i: (i, 0)),
                  pl.BlockSpec((L2, bo), lambda j, i: (0, j)),
                  pl.BlockSpec((bn, L1), lambda j, i: (i, 0)),
                  pl.BlockSpec((L1, bo), lambda j, i: (0, j)),
                  pl.BlockSpec((1, bo), lambda j, i: (0, j)),
                  pl.BlockSpec((1, bo), lambda j, i: (0, j)),
                  pl.BlockSpec((1, 1, bn), lambda j, i: (i, 0, 0))],
        out_specs=pl.BlockSpec((NUM_GRAPHS, bo), lambda j, i: (0, j)),
        out_shape=jax.ShapeDtypeStruct((NUM_GRAPHS, L3), jnp.float32),
    )(x3p, x2p, p["skip23"]["W"], x1p, p["skip13"]["W"],
      p["skip23"]["b"][None, :], p["skip13"]["b"][None, :], batch3d)


def _head_body(pooled_ref, w1_ref, b1_ref, w2_ref, b2_ref, out_ref):
    f = jax.nn.relu(_dot(pooled_ref[...], w1_ref[...]) + b1_ref[...])
    out_ref[...] = _dot(f, w2_ref[...]) + b2_ref[...]


def _head(pooled, p):
    return pl.pallas_call(
        _head_body,
        out_shape=jax.ShapeDtypeStruct((NUM_GRAPHS, NUM_CLASSES), jnp.float32),
    )(pooled, p["fc1"]["W"], p["fc1"]["b"][None, :],
      p["fc2"]["W"], p["fc2"]["b"][None, :])


_CONV_CFG = {
    1: dict(i_dim=L0, o_dim=L1, bn=2000, bo=512, abe=2000, abo=512,
            mbn=512, mbk=2048, nch=4),
    2: dict(i_dim=L1, o_dim=L2, bn=400, bo=512, abe=800, abo=512,
            mbn=512, mbk=2048, nch=4),
    3: dict(i_dim=L2, o_dim=L3, bn=400, bo=512, abe=800, abo=512,
            mbn=512, mbk=1024, nch=4),
}


def kernel(x, edge_index, batch, params):
    p = params
    src = edge_index[0]
    dst = edge_index[1]

    feats, h = _node_encoder(x, p)
    fa = jnp.take(feats, src, axis=0)
    fb = jnp.take(feats, dst, axis=0)
    w = _edge_mlp(fa, fb, p)

    def conv(hk, c, cfg):
        xl, xr = _dense2(hk, c, cfg["i_dim"], cfg["o_dim"], cfg["bn"],
                         cfg["bo"])
        nch = cfg["nch"]
        ec = E // nch
        parts = []
        for t in range(nch):
            sl = slice(t * ec, (t + 1) * ec)
            xlg = jnp.take(xl, src[sl], axis=0)
            xrg = jnp.take(xr, dst[sl], axis=0)
            parts.append(_alpha(xlg, xrg, w[sl], c, cfg["o_dim"],
                                cfg["abe"], cfg["abo"]))
        alpha = jnp.concatenate(parts)
        amax = jax.ops.segment_max(alpha, dst, num_segments=N)
        amax = jnp.where(jnp.isfinite(amax), amax, 0.0)
        ex = jnp.exp(alpha - amax[dst])
        lin = dst * NP + src
        a_mat = jnp.zeros((NP * NP,), jnp.float32).at[lin].add(ex).reshape(NP, NP)
        xlp = jnp.pad(xl, ((0, NP - N), (0, 0)))
        return _aggmm(a_mat, xlp, cfg["o_dim"], cfg["mbn"], cfg["mbk"])[:N]

    r1 = conv(h, p["conv1"], _CONV_CFG[1])
    x1p = _lnleaky(r1, p["conv1"]["bias"], p["ln1"]["g"], p["ln1"]["b"],
                   L1, 2000)
    r2 = conv(x1p, p["conv2"], _CONV_CFG[2])
    x2p = _lnleaky(r2, p["conv2"]["bias"], p["ln2"]["g"], p["ln2"]["b"],
                   L2, 1000)
    r3 = conv(x2p, p["conv3"], _CONV_CFG[3])
    x3p = _lnleaky(r3, p["conv3"]["bias"], p["ln3"]["g"], p["ln3"]["b"],
                   L3, 400)

    batch3d = batch.reshape(N // 1000, 1, 1000)
    pooled = _xfpool(x3p, x2p, x1p, batch3d, p, 1000, 512)
    return _head(pooled, p)
